# initial kernel scaffold (unmeasured)
import jax
import jax.numpy as jnp
from jax import lax
from jax.experimental import pallas as pl
from jax.experimental.pallas import tpu as pltpu

N_DEV = 32
N_TOK = 512
D_IN = 256
D_OUT = 512
N_EXP = 64
EXP_PER_DEV = 2
CHUNK = N_TOK // N_DEV


def kernel(x, router_W, route_idx, expert_W):
    def body(x_ref, rw_ref, idx_ref, ew_ref, out_ref,
             rs_buf, ag_buf, rs_send, rs_recv, ag_send, ag_recv):
        me = lax.axis_index("i")

        xv = x_ref[:, :]
        scores = jnp.dot(xv, rw_ref[:, :], preferred_element_type=jnp.float32)
        smax = jnp.max(scores, axis=-1, keepdims=True)
        p = jnp.exp(scores - smax)
        probs = p / jnp.sum(p, axis=-1, keepdims=True)

        e_ids = lax.broadcasted_iota(jnp.int32, (N_TOK, N_EXP), 1)
        i0 = idx_ref[:, 0:1]
        i1 = idx_ref[:, 1:2]
        g0 = jnp.sum(jnp.where(e_ids == i0, probs, 0.0), axis=-1, keepdims=True)
        g1 = jnp.sum(jnp.where(e_ids == i1, probs, 0.0), axis=-1, keepdims=True)
        gs = g0 + g1

        partial = jnp.zeros((N_TOK, D_OUT), jnp.float32)
        for l in range(EXP_PER_DEV):
            ge = me * EXP_PER_DEV + l
            chosen = jnp.logical_or(i0 == ge, i1 == ge)
            p_ge = jnp.sum(jnp.where(e_ids == ge, probs, 0.0),
                           axis=-1, keepdims=True)
            w = jnp.where(chosen, p_ge / gs, 0.0)
            partial = partial + jnp.dot(
                xv * w, ew_ref[l], preferred_element_type=jnp.float32)
        out_ref[:, :] = partial

        rs = []
        for s in range(1, N_DEV):
            tgt = (me + s) % N_DEV
            r = pltpu.make_async_remote_copy(
                src_ref=out_ref.at[pl.ds(tgt * CHUNK, CHUNK), :],
                dst_ref=rs_buf.at[s],
                send_sem=rs_send.at[s],
                recv_sem=rs_recv.at[s],
                device_id=(tgt,),
                device_id_type=pl.DeviceIdType.MESH,
            )
            r.start()
            rs.append(r)
        for r in rs:
            r.wait_recv()
        for r in rs:
            r.wait_send()

        rs_buf[0, :, :] = out_ref[pl.ds(me * CHUNK, CHUNK), :]
        red = jnp.sum(rs_buf[:, :, :], axis=0)
        ag_buf[:, :] = red

        ag = []
        for s in range(1, N_DEV):
            tgt = (me + s) % N_DEV
            r = pltpu.make_async_remote_copy(
                src_ref=ag_buf,
                dst_ref=out_ref.at[pl.ds(me * CHUNK, CHUNK), :],
                send_sem=ag_send.at[s],
                recv_sem=ag_recv.at[s],
                device_id=(tgt,),
                device_id_type=pl.DeviceIdType.MESH,
            )
            r.start()
            ag.append(r)
        out_ref[pl.ds(me * CHUNK, CHUNK), :] = red
        for r in ag:
            r.wait_recv()
        for r in ag:
            r.wait_send()

    return pl.pallas_call(
        body,
        out_shape=jax.ShapeDtypeStruct((N_TOK, D_OUT), jnp.float32),
        in_specs=[pl.BlockSpec(memory_space=pltpu.VMEM)] * 4,
        out_specs=pl.BlockSpec(memory_space=pltpu.VMEM),
        scratch_shapes=[
            pltpu.VMEM((N_DEV, CHUNK, D_OUT), jnp.float32),
            pltpu.VMEM((CHUNK, D_OUT), jnp.float32),
            pltpu.SemaphoreType.DMA((N_DEV,)),
            pltpu.SemaphoreType.DMA((N_DEV,)),
            pltpu.SemaphoreType.DMA((N_DEV,)),
            pltpu.SemaphoreType.DMA((N_DEV,)),
        ],
        compiler_params=pltpu.CompilerParams(collective_id=0),
    )(x, router_W, route_idx, expert_W)


# baseline (device time: 48018 ns/iter reference)
import jax
import jax.numpy as jnp
from jax import lax
from jax.experimental import pallas as pl
from jax.experimental.pallas import tpu as pltpu

N_DEV = 32
N_TOK = 512
D_IN = 256
D_OUT = 512
N_EXP = 64
EXP_PER_DEV = 2
CHUNK = N_TOK // N_DEV


def kernel(x, router_W, route_idx, expert_W):
    def body(x_ref, rw_ref, idx_ref, ew_ref, out_ref,
             rs_buf, ag_buf, rs_send, rs_recv, ag_send, ag_recv):
        me = lax.axis_index("i")

        xv = x_ref[:, :]
        scores = jnp.dot(xv, rw_ref[:, :], preferred_element_type=jnp.float32)
        smax = jnp.max(scores, axis=-1, keepdims=True)
        p = jnp.exp(scores - smax)
        probs = p / jnp.sum(p, axis=-1, keepdims=True)

        e_ids = lax.broadcasted_iota(jnp.int32, (N_TOK, N_EXP), 1)
        i0 = idx_ref[:, 0:1]
        i1 = idx_ref[:, 1:2]
        g0 = jnp.sum(jnp.where(e_ids == i0, probs, 0.0), axis=-1, keepdims=True)
        g1 = jnp.sum(jnp.where(e_ids == i1, probs, 0.0), axis=-1, keepdims=True)
        gs = g0 + g1

        partial = jnp.zeros((N_TOK, D_OUT), jnp.float32)
        for l in range(EXP_PER_DEV):
            ge = me * EXP_PER_DEV + l
            chosen = jnp.logical_or(i0 == ge, i1 == ge)
            p_ge = jnp.sum(jnp.where(e_ids == ge, probs, 0.0),
                           axis=-1, keepdims=True)
            w = jnp.where(chosen, p_ge / gs, 0.0)
            partial = partial + jnp.dot(
                xv * w, ew_ref[l], preferred_element_type=jnp.float32)
        out_ref[:, :] = partial

        rs = []
        for s in range(1, N_DEV):
            tgt = (me + s) % N_DEV
            r = pltpu.make_async_remote_copy(
                src_ref=out_ref.at[pl.ds(tgt * CHUNK, CHUNK), :],
                dst_ref=rs_buf.at[s],
                send_sem=rs_send.at[s],
                recv_sem=rs_recv.at[s],
                device_id=(tgt,),
                device_id_type=pl.DeviceIdType.MESH,
            )
            r.start()
            rs.append(r)
        for r in rs:
            r.wait_recv()
        for r in rs:
            r.wait_send()

        rs_buf[0, :, :] = out_ref[pl.ds(me * CHUNK, CHUNK), :]
        red = jnp.sum(rs_buf[:, :, :], axis=0)
        ag_buf[:, :] = red

        ag = []
        for s in range(1, N_DEV):
            tgt = (me + s) % N_DEV
            r = pltpu.make_async_remote_copy(
                src_ref=ag_buf,
                dst_ref=out_ref.at[pl.ds(me * CHUNK, CHUNK), :],
                send_sem=ag_send.at[s],
                recv_sem=ag_recv.at[s],
                device_id=(tgt,),
                device_id_type=pl.DeviceIdType.MESH,
            )
            r.start()
            ag.append(r)
        out_ref[pl.ds(me * CHUNK, CHUNK), :] = red
        for r in ag:
            r.wait_recv()
        for r in ag:
            r.wait_send()

    return pl.pallas_call(
        body,
        out_shape=jax.ShapeDtypeStruct((N_TOK, D_OUT), jnp.float32),
        in_specs=[pl.BlockSpec(memory_space=pltpu.VMEM)] * 4,
        out_specs=pl.BlockSpec(memory_space=pltpu.VMEM),
        scratch_shapes=[
            pltpu.VMEM((N_DEV, CHUNK, D_OUT), jnp.float32),
            pltpu.VMEM((CHUNK, D_OUT), jnp.float32),
            pltpu.SemaphoreType.DMA((N_DEV,)),
            pltpu.SemaphoreType.DMA((N_DEV,)),
            pltpu.SemaphoreType.DMA((N_DEV,)),
            pltpu.SemaphoreType.DMA((N_DEV,)),
        ],
    )(x, router_W, route_idx, expert_W)
